# baseline (speedup 1.0000x reference)
import functools

import jax
import jax.numpy as jnp
from jax import lax
from jax.experimental import pallas as pl
from jax.experimental.pallas import tpu as pltpu
from jax.experimental.pallas import tpu_sc as plsc

NI = 10000
NU = 10000
D = 128
E = 320000
NROWS = NI + NU
B_SCORE = 8192

NC = 2
NS = 16
IDXW = 128
CW = 64
NBUF = 4
E_PAD = 327680
EPT = E_PAD // NS
NCHUNK = EPT // CW
NB = 32
ACCN = NI + 16

NSC_SCORE = 2 * 2 * B_SCORE
SPT = NSC_SCORE // (NC * NS)
NSCHUNK = SPT // IDXW

_mesh = plsc.VectorSubcoreMesh(
    core_axis_name="c", subcore_axis_name="s", num_cores=NC, num_subcores=NS)


def _msg_body(with_deg, *refs):
    if with_deg:
        (h_hbm, src_hbm, dst_hbm, z2_hbm, z1_hbm, ones_hbm,
         msg_hbm, deg_hbm, sidx, didx, *rest) = refs
        rows = rest[:NBUF]
        ones, dtmp, acc, dacc = rest[NBUF:NBUF + 4]
        sems = rest[NBUF + 4:]
    else:
        (h_hbm, src_hbm, dst_hbm, z2_hbm,
         msg_hbm, sidx, didx, *rest) = refs
        rows = rest[:NBUF]
        acc = rest[NBUF]
        sems = rest[NBUF + 1:]
    cid = lax.axis_index("c")
    sid = lax.axis_index("s")
    irow = cid * (E_PAD // CW) + sid * NCHUNK

    @pl.when(sid < 10)
    def _():
        pltpu.sync_copy(z2_hbm.at[pl.ds(sid * 1000, 1000)],
                        acc.at[pl.ds(sid * 1000, 1000)])
        if with_deg:
            pltpu.sync_copy(z1_hbm.at[pl.ds(sid * 1000, 1000)], dtmp)
            pltpu.sync_copy(dtmp, dacc.at[pl.ds(sid * 1000, 1000)])

    if with_deg:
        pltpu.sync_copy(ones_hbm, ones)
    plsc.subcore_barrier()

    def group(g, carry):
        pltpu.sync_copy(src_hbm.at[pl.ds(irow + g * NB, NB)], sidx)
        pltpu.sync_copy(dst_hbm.at[pl.ds(irow + g * NB, NB)], didx)
        for b in range(NBUF):
            pltpu.async_copy(h_hbm.at[sidx.at[b]], rows[b], sems[b])

        def quad(q, c):
            for b in range(NBUF):
                a = q * NBUF + b
                pltpu.make_async_copy(h_hbm.at[sidx.at[a]], rows[b],
                                      sems[b]).wait()
                pltpu.sync_copy(rows[b], acc.at[didx.at[a]], add=True)
                if with_deg:
                    pltpu.sync_copy(ones, dacc.at[didx.at[a]], add=True)

                @pl.when(q < NB // NBUF - 1)
                def _():
                    pltpu.async_copy(h_hbm.at[sidx.at[a + NBUF]], rows[b],
                                     sems[b])
            return c

        lax.fori_loop(0, NB // NBUF, quad, 0)
        return carry

    lax.fori_loop(0, NCHUNK // NB, group, 0)
    plsc.subcore_barrier()

    @pl.when(sid < 10)
    def _():
        pltpu.sync_copy(acc.at[pl.ds(sid * 1000, 1000)],
                        msg_hbm.at[pl.ds(cid * NI + sid * 1000, 1000)])
        if with_deg:
            pltpu.sync_copy(dacc.at[pl.ds(sid * 1000, 1000)], dtmp)
            pltpu.sync_copy(dtmp, deg_hbm.at[pl.ds(cid * NI + sid * 1000, 1000)])


def _make_msg_kernel(with_deg):
    out_type = [jax.ShapeDtypeStruct((NROWS, D), jnp.float32)]
    scratch = [
        pltpu.VMEM((NB, CW), jnp.int32),
        pltpu.VMEM((NB, CW), jnp.int32),
    ]
    scratch += [pltpu.VMEM((CW, D), jnp.float32) for _ in range(NBUF)]
    if with_deg:
        out_type.append(jax.ShapeDtypeStruct((NROWS,), jnp.float32))
        scratch.append(pltpu.VMEM((CW,), jnp.float32))
        scratch.append(pltpu.VMEM((1000,), jnp.float32))
    scratch.append(pltpu.VMEM_SHARED((ACCN, D), jnp.float32))
    if with_deg:
        scratch.append(pltpu.VMEM_SHARED((ACCN,), jnp.float32))
    scratch += [pltpu.SemaphoreType.DMA for _ in range(NBUF)]
    return pl.kernel(functools.partial(_msg_body, with_deg),
                     out_type=out_type, mesh=_mesh, scratch_types=scratch)


def _gather_body(h_hbm, idx_hbm, out_hbm, idx_v, rows, sem):
    cid = lax.axis_index("c")
    sid = lax.axis_index("s")
    wid = sid * NC + cid

    def chunk(j, carry):
        g = wid * SPT + j * IDXW
        pltpu.sync_copy(idx_hbm.at[pl.ds(g, IDXW)], idx_v)
        pltpu.async_copy(h_hbm.at[idx_v], rows, sem).wait()
        pltpu.sync_copy(rows, out_hbm.at[pl.ds(g, IDXW)])
        return carry

    lax.fori_loop(0, NSCHUNK, chunk, 0)


_gather_kernel = pl.kernel(
    _gather_body,
    out_type=jax.ShapeDtypeStruct((NSC_SCORE, D), jnp.float32),
    mesh=_mesh,
    scratch_types=[
        pltpu.VMEM((IDXW,), jnp.int32),
        pltpu.VMEM((IDXW, D), jnp.float32),
        pltpu.SemaphoreType.DMA,
    ])


_BLK = 400
_NBLK = NROWS // _BLK
_HALF = NI // _BLK


def _proj_body(x_ref, w_ref, o_ref):
    o_ref[...] = jnp.dot(x_ref[...], w_ref[0],
                         preferred_element_type=jnp.float32)


_proj = pl.pallas_call(
    _proj_body,
    grid=(_NBLK,),
    in_specs=[
        pl.BlockSpec((_BLK, D), lambda i: (i, 0)),
        pl.BlockSpec((1, D, D), lambda i: (i // _HALF, 0, 0)),
    ],
    out_specs=pl.BlockSpec((_BLK, D), lambda i: (i, 0)),
    out_shape=jax.ShapeDtypeStruct((NROWS, D), jnp.float32),
)


def _layer_body(residual, *refs):
    if residual:
        h_ref, m_ref, deg_ref, ws_ref, wn_ref, p_ref, o_ref = refs
    else:
        h_ref, m_ref, deg_ref, ws_ref, wn_ref, o_ref = refs
    inv = 1.0 / jnp.maximum(deg_ref[...], 1.0)
    msg = m_ref[...] * inv
    z = (jnp.dot(h_ref[...], ws_ref[0], preferred_element_type=jnp.float32)
         + jnp.dot(msg, wn_ref[0], preferred_element_type=jnp.float32))
    z = jnp.maximum(z, 0.0)
    nrm = jnp.sqrt(jnp.sum(z * z, axis=1, keepdims=True))
    z = z / jnp.maximum(nrm, 1e-6)
    if residual:
        z = z + p_ref[...]
    o_ref[...] = z


def _make_layer(residual):
    in_specs = [
        pl.BlockSpec((_BLK, D), lambda i: (i, 0)),
        pl.BlockSpec((_BLK, D), lambda i: (i, 0)),
        pl.BlockSpec((_BLK, 1), lambda i: (i, 0)),
        pl.BlockSpec((1, D, D), lambda i: (i // _HALF, 0, 0)),
        pl.BlockSpec((1, D, D), lambda i: (i // _HALF, 0, 0)),
    ]
    if residual:
        in_specs.append(pl.BlockSpec((_BLK, D), lambda i: (i, 0)))
    return pl.pallas_call(
        functools.partial(_layer_body, residual),
        grid=(_NBLK,),
        in_specs=in_specs,
        out_specs=pl.BlockSpec((_BLK, D), lambda i: (i, 0)),
        out_shape=jax.ShapeDtypeStruct((NROWS, D), jnp.float32),
    )


_SBLK = 512
_SNBLK = (2 * B_SCORE) // _SBLK


def _dot_body(u_ref, i_ref, o_ref):
    o_ref[...] = jnp.sum(u_ref[...] * i_ref[...], axis=1, keepdims=True)


_dot = pl.pallas_call(
    _dot_body,
    grid=(_SNBLK,),
    in_specs=[
        pl.BlockSpec((_SBLK, D), lambda i: (i, 0)),
        pl.BlockSpec((_SBLK, D), lambda i: (i + _SNBLK, 0)),
    ],
    out_specs=pl.BlockSpec((_SBLK, 1), lambda i: (i, 0)),
    out_shape=jax.ShapeDtypeStruct((2 * B_SCORE, 1), jnp.float32),
)


def kernel(x_item, x_user, W_item, W_user, Ws_item, Wn_item, Ws_user, Wn_user,
           edge_index, pos_edges, neg_edges):
    eu = edge_index[0].astype(jnp.int32)
    ei = edge_index[1].astype(jnp.int32)
    npad = E_PAD - E
    pad_src = jnp.zeros((npad,), jnp.int32)
    pad_dst = jnp.full((npad,), NI, jnp.int32)
    src_idx = jnp.concatenate([eu + NI, pad_src,
                               ei + NROWS, pad_src]).reshape(-1, CW)
    dst_idx = jnp.concatenate([ei, pad_dst, eu, pad_dst]).reshape(-1, CW)

    x_all = jnp.concatenate([x_item, x_user], axis=0)
    Wp = jnp.stack([W_item, W_user])
    z2 = jnp.zeros((NI, D), jnp.float32)
    z1 = jnp.zeros((NI,), jnp.float32)
    ones = jnp.ones((CW,), jnp.float32)

    p_all = _proj(x_all, Wp)

    msg_deg = _make_msg_kernel(True)
    msg_only = _make_msg_kernel(False)

    msg0, deg = msg_deg(jnp.concatenate([p_all, p_all]), src_idx, dst_idx,
                        z2, z1, ones)
    deg2d = deg.reshape(NROWS, 1)

    Ws0 = jnp.stack([Ws_item[0], Ws_user[0]])
    Wn0 = jnp.stack([Wn_item[0], Wn_user[0]])
    Ws1 = jnp.stack([Ws_item[1], Ws_user[1]])
    Wn1 = jnp.stack([Wn_item[1], Wn_user[1]])

    h1 = _make_layer(False)(p_all, msg0, deg2d, Ws0, Wn0)
    (msg1,) = msg_only(jnp.concatenate([h1, h1]), src_idx, dst_idx, z2)
    h2 = _make_layer(True)(h1, msg1, deg2d, Ws1, Wn1, p_all)

    su = jnp.concatenate([pos_edges[0], neg_edges[0]]).astype(jnp.int32) + NI
    si = jnp.concatenate([pos_edges[1], neg_edges[1]]).astype(jnp.int32)
    gidx = jnp.concatenate([su, si])
    grows = _gather_kernel(h2, gidx)
    scores = _dot(grows, grows)[:, 0]
    return scores[:B_SCORE], scores[B_SCORE:]


# speedup vs baseline: 1.0211x; 1.0211x over previous
import functools

import jax
import jax.numpy as jnp
from jax import lax
from jax.experimental import pallas as pl
from jax.experimental.pallas import tpu as pltpu
from jax.experimental.pallas import tpu_sc as plsc

NI = 10000
NU = 10000
D = 128
E = 320000
NROWS = NI + NU
B_SCORE = 8192

NC = 2
NS = 16
IDXW = 128
CW = 64
NBUF = 4
E_PAD = 327680
EPT = E_PAD // NS
NCHUNK = EPT // CW
NB = 32
ACCN = NI + 16

NSC_SCORE = 2 * 2 * B_SCORE
SPT = NSC_SCORE // (NC * NS)
NSCHUNK = SPT // IDXW

_mesh = plsc.VectorSubcoreMesh(
    core_axis_name="c", subcore_axis_name="s", num_cores=NC, num_subcores=NS)


def _msg_body(with_deg, *refs):
    if with_deg:
        (h_hbm, src_hbm, dst_hbm, z2_hbm, z1_hbm, ones_hbm,
         msg_hbm, deg_hbm, sidx, didx, *rest) = refs
        rows = rest[:NBUF]
        ones, dtmp, acc, dacc = rest[NBUF:NBUF + 4]
        sems = rest[NBUF + 4:]
    else:
        (h_hbm, src_hbm, dst_hbm, z2_hbm,
         msg_hbm, sidx, didx, *rest) = refs
        rows = rest[:NBUF]
        acc = rest[NBUF]
        sems = rest[NBUF + 1:]
    cid = lax.axis_index("c")
    sid = lax.axis_index("s")
    irow = cid * (E_PAD // CW) + sid * NCHUNK

    @pl.when(sid < 10)
    def _():
        pltpu.sync_copy(z2_hbm.at[pl.ds(sid * 1000, 1000)],
                        acc.at[pl.ds(sid * 1000, 1000)])
        if with_deg:
            pltpu.sync_copy(z1_hbm.at[pl.ds(sid * 1000, 1000)], dtmp)
            pltpu.sync_copy(dtmp, dacc.at[pl.ds(sid * 1000, 1000)])

    if with_deg:
        pltpu.sync_copy(ones_hbm, ones)
    plsc.subcore_barrier()

    def group(g, carry):
        pltpu.sync_copy(src_hbm.at[pl.ds(irow + g * NB, NB)], sidx)
        pltpu.sync_copy(dst_hbm.at[pl.ds(irow + g * NB, NB)], didx)
        for b in range(NBUF):
            pltpu.async_copy(h_hbm.at[sidx.at[b]], rows[b], sems[b])

        def quad(q, c):
            for b in range(NBUF):
                a = q * NBUF + b
                pltpu.make_async_copy(h_hbm.at[sidx.at[a]], rows[b],
                                      sems[b]).wait()
                pltpu.sync_copy(rows[b], acc.at[didx.at[a]], add=True)
                if with_deg:
                    pltpu.sync_copy(ones, dacc.at[didx.at[a]], add=True)

                @pl.when(q < NB // NBUF - 1)
                def _():
                    pltpu.async_copy(h_hbm.at[sidx.at[a + NBUF]], rows[b],
                                     sems[b])
            return c

        lax.fori_loop(0, NB // NBUF, quad, 0)
        return carry

    lax.fori_loop(0, NCHUNK // NB, group, 0)
    plsc.subcore_barrier()

    @pl.when(sid < 10)
    def _():
        pltpu.sync_copy(acc.at[pl.ds(sid * 1000, 1000)],
                        msg_hbm.at[pl.ds(cid * NI + sid * 1000, 1000)])
        if with_deg:
            pltpu.sync_copy(dacc.at[pl.ds(sid * 1000, 1000)], dtmp)
            pltpu.sync_copy(dtmp, deg_hbm.at[pl.ds(cid * NI + sid * 1000, 1000)])


def _make_msg_kernel(with_deg):
    out_type = [jax.ShapeDtypeStruct((NROWS, D), jnp.float32)]
    scratch = [
        pltpu.VMEM((NB, CW), jnp.int32),
        pltpu.VMEM((NB, CW), jnp.int32),
    ]
    scratch += [pltpu.VMEM((CW, D), jnp.float32) for _ in range(NBUF)]
    if with_deg:
        out_type.append(jax.ShapeDtypeStruct((NROWS,), jnp.float32))
        scratch.append(pltpu.VMEM((CW,), jnp.float32))
        scratch.append(pltpu.VMEM((1000,), jnp.float32))
    scratch.append(pltpu.VMEM_SHARED((ACCN, D), jnp.float32))
    if with_deg:
        scratch.append(pltpu.VMEM_SHARED((ACCN,), jnp.float32))
    scratch += [pltpu.SemaphoreType.DMA for _ in range(NBUF)]
    return pl.kernel(functools.partial(_msg_body, with_deg),
                     out_type=out_type, mesh=_mesh, scratch_types=scratch)


def _gather_body(h_hbm, idx_hbm, out_hbm, idx_v, rows, sem):
    cid = lax.axis_index("c")
    sid = lax.axis_index("s")
    wid = sid * NC + cid

    def chunk(j, carry):
        g = wid * SPT + j * IDXW
        pltpu.sync_copy(idx_hbm.at[pl.ds(g, IDXW)], idx_v)
        pltpu.async_copy(h_hbm.at[idx_v], rows, sem).wait()
        pltpu.sync_copy(rows, out_hbm.at[pl.ds(g, IDXW)])
        return carry

    lax.fori_loop(0, NSCHUNK, chunk, 0)


_gather_kernel = pl.kernel(
    _gather_body,
    out_type=jax.ShapeDtypeStruct((NSC_SCORE, D), jnp.float32),
    mesh=_mesh,
    scratch_types=[
        pltpu.VMEM((IDXW,), jnp.int32),
        pltpu.VMEM((IDXW, D), jnp.float32),
        pltpu.SemaphoreType.DMA,
    ])


_BLK = 400
_NBLK = NROWS // _BLK
_HALF = NI // _BLK


def _proj_body(x_ref, w_ref, o_ref):
    o_ref[...] = jnp.dot(x_ref[...], w_ref[0],
                         preferred_element_type=jnp.float32)


_proj = pl.pallas_call(
    _proj_body,
    grid=(_NBLK,),
    in_specs=[
        pl.BlockSpec((_BLK, D), lambda i: (i, 0)),
        pl.BlockSpec((1, D, D), lambda i: (i // _HALF, 0, 0)),
    ],
    out_specs=pl.BlockSpec((_BLK, D), lambda i: (i, 0)),
    out_shape=jax.ShapeDtypeStruct((NROWS, D), jnp.float32),
)


def _layer_body(residual, *refs):
    if residual:
        h_ref, m_ref, deg_ref, ws_ref, wn_ref, p_ref, o_ref = refs
    else:
        h_ref, m_ref, deg_ref, ws_ref, wn_ref, o_ref = refs
    inv = 1.0 / jnp.maximum(deg_ref[...], 1.0)
    msg = m_ref[...] * inv
    z = (jnp.dot(h_ref[...], ws_ref[0], preferred_element_type=jnp.float32)
         + jnp.dot(msg, wn_ref[0], preferred_element_type=jnp.float32))
    z = jnp.maximum(z, 0.0)
    nrm = jnp.sqrt(jnp.sum(z * z, axis=1, keepdims=True))
    z = z / jnp.maximum(nrm, 1e-6)
    if residual:
        z = z + p_ref[...]
    o_ref[...] = z


def _make_layer(residual):
    in_specs = [
        pl.BlockSpec((_BLK, D), lambda i: (i, 0)),
        pl.BlockSpec((_BLK, D), lambda i: (i, 0)),
        pl.BlockSpec((_BLK, 1), lambda i: (i, 0)),
        pl.BlockSpec((1, D, D), lambda i: (i // _HALF, 0, 0)),
        pl.BlockSpec((1, D, D), lambda i: (i // _HALF, 0, 0)),
    ]
    if residual:
        in_specs.append(pl.BlockSpec((_BLK, D), lambda i: (i, 0)))
    return pl.pallas_call(
        functools.partial(_layer_body, residual),
        grid=(_NBLK,),
        in_specs=in_specs,
        out_specs=pl.BlockSpec((_BLK, D), lambda i: (i, 0)),
        out_shape=jax.ShapeDtypeStruct((NROWS, D), jnp.float32),
    )


_SBLK = 512
_SNBLK = (2 * B_SCORE) // _SBLK


def _dot_body(u_ref, i_ref, o_ref):
    o_ref[...] = jnp.sum(u_ref[...] * i_ref[...], axis=1, keepdims=True)


_dot = pl.pallas_call(
    _dot_body,
    grid=(_SNBLK,),
    in_specs=[
        pl.BlockSpec((_SBLK, D), lambda i: (i, 0)),
        pl.BlockSpec((_SBLK, D), lambda i: (i + _SNBLK, 0)),
    ],
    out_specs=pl.BlockSpec((_SBLK, 1), lambda i: (i, 0)),
    out_shape=jax.ShapeDtypeStruct((2 * B_SCORE, 1), jnp.float32),
)


def kernel(x_item, x_user, W_item, W_user, Ws_item, Wn_item, Ws_user, Wn_user,
           edge_index, pos_edges, neg_edges):
    eu = edge_index[0].astype(jnp.int32)
    ei = edge_index[1].astype(jnp.int32)
    npad = E_PAD - E
    pad_src = jnp.zeros((npad,), jnp.int32)
    pad_dst = jnp.full((npad,), NI, jnp.int32)
    src_idx = jnp.concatenate([eu + NI, pad_src, ei, pad_src]).reshape(-1, CW)
    dst_idx = jnp.concatenate([ei, pad_dst, eu, pad_dst]).reshape(-1, CW)

    x_all = jnp.concatenate([x_item, x_user], axis=0)
    Wp = jnp.stack([W_item, W_user])
    z2 = jnp.zeros((NI, D), jnp.float32)
    z1 = jnp.zeros((NI,), jnp.float32)
    ones = jnp.ones((CW,), jnp.float32)

    p_all = _proj(x_all, Wp)

    msg_deg = _make_msg_kernel(True)
    msg_only = _make_msg_kernel(False)

    msg0, deg = msg_deg(p_all, src_idx, dst_idx, z2, z1, ones)
    deg2d = deg.reshape(NROWS, 1)

    Ws0 = jnp.stack([Ws_item[0], Ws_user[0]])
    Wn0 = jnp.stack([Wn_item[0], Wn_user[0]])
    Ws1 = jnp.stack([Ws_item[1], Ws_user[1]])
    Wn1 = jnp.stack([Wn_item[1], Wn_user[1]])

    h1 = _make_layer(False)(p_all, msg0, deg2d, Ws0, Wn0)
    (msg1,) = msg_only(h1, src_idx, dst_idx, z2)
    h2 = _make_layer(True)(h1, msg1, deg2d, Ws1, Wn1, p_all)

    su = jnp.concatenate([pos_edges[0], neg_edges[0]]).astype(jnp.int32) + NI
    si = jnp.concatenate([pos_edges[1], neg_edges[1]]).astype(jnp.int32)
    gidx = jnp.concatenate([su, si])
    grows = _gather_kernel(h2, gidx)
    scores = _dot(grows, grows)[:, 0]
    return scores[:B_SCORE], scores[B_SCORE:]
